# baseline (speedup 1.0000x reference)
import functools

import jax
import jax.numpy as jnp
from jax import lax
from jax.experimental import pallas as pl
from jax.experimental.pallas import tpu as pltpu
from jax.experimental.pallas import tpu_sc as plsc

VOCAB = 100000
EMBED = 64
MAX_NORM = 1.0
B = 1024
L = 20
N_ROWS = B * L

_NC, _NS = 2, 16
NW = _NC * _NS
ROWS_PER_W = N_ROWS // NW
CHUNK = 128
N_CHUNKS = ROWS_PER_W // CHUNK

MB = 32
GRID_B = B // MB
VN = 4096
NFULL = VOCAB // VN
VTAIL = VOCAB - NFULL * VN
VTAIL_PAD = 1792
WPAD = NFULL * VN + VTAIL_PAD
NCH = 4
RCH = MB // NCH


@functools.lru_cache(maxsize=1)
def _make_sc_gather():
    @functools.partial(
        pl.kernel,
        mesh=plsc.VectorSubcoreMesh(core_axis_name="c", subcore_axis_name="s"),
        out_type=jax.ShapeDtypeStruct((N_ROWS, EMBED), jnp.float32),
        scratch_types=[
            pltpu.VMEM((N_CHUNKS, CHUNK), jnp.int32),
            pltpu.VMEM((ROWS_PER_W, EMBED), jnp.float32),
            pltpu.SemaphoreType.DMA,
        ],
        compiler_params=pltpu.CompilerParams(use_tc_tiling_on_sc=False),
    )
    def _sc_gather(idx_hbm, table_hbm, emb_hbm, idx_v, rows_v, sem):
        wid = lax.axis_index("s") * _NC + lax.axis_index("c")
        pltpu.sync_copy(idx_hbm.at[wid], idx_v)
        copies = []
        for k in range(N_CHUNKS):
            copies.append(
                pltpu.async_copy(
                    table_hbm.at[idx_v.at[k]],
                    rows_v.at[pl.ds(k * CHUNK, CHUNK)],
                    sem,
                )
            )
        for c in copies:
            c.wait()
        pltpu.sync_copy(rows_v, emb_hbm.at[pl.ds(wid * ROWS_PER_W, ROWS_PER_W)])

    return _sc_gather


def _mm_body(emb_ref, wb_ref, b_ref, out_hbm, out_vmem, sems):
    j = pl.program_id(0)

    e = emb_ref[pl.ds(j * MB * L, MB * L), :]
    ss = jnp.sum(e * e, axis=1, keepdims=True)
    scale = jnp.minimum(1.0, MAX_NORM / jnp.maximum(jnp.sqrt(ss), 1e-7))
    x = jnp.sum((e * scale).reshape(MB, L, EMBED), axis=1) * (1.0 / L)
    xb = x.astype(jnp.bfloat16)

    def row_copy(buf, c, row0):
        return pltpu.make_async_copy(
            out_vmem.at[buf, pl.ds(c * RCH, RCH), :],
            out_hbm.at[pl.ds(row0 + c * RCH, RCH), :],
            sems.at[buf, c],
        )

    for buf in range(2):
        @pl.when(lax.rem(j, 2) == buf)
        def _(buf=buf):
            @pl.when(j >= 2)
            def _():
                for c in range(NCH):
                    row_copy(buf, c, (j - 2) * MB).wait()

            for n in range(NFULL):
                wv = wb_ref[pl.ds(n * VN, VN), :]
                acc = lax.dot_general(
                    xb, wv, (((1,), (1,)), ((), ())),
                    preferred_element_type=jnp.float32,
                )
                out_vmem[buf, :, pl.ds(n * VN, VN)] = acc + b_ref[0, pl.ds(n * VN, VN)]
            wv = wb_ref[pl.ds(NFULL * VN, VTAIL_PAD), :]
            acc = lax.dot_general(
                xb, wv, (((1,), (1,)), ((), ())),
                preferred_element_type=jnp.float32,
            )[:, :VTAIL]
            out_vmem[buf, :, pl.ds(NFULL * VN, VTAIL)] = (
                acc + b_ref[0, pl.ds(NFULL * VN, VTAIL)]
            )

            for c in range(NCH):
                row_copy(buf, c, j * MB).start()

    @pl.when(j == GRID_B - 1)
    def _():
        last_buf = (GRID_B - 1) % 2
        for c in range(NCH):
            row_copy(1 - last_buf, c, (GRID_B - 2) * MB).wait()
        for c in range(NCH):
            row_copy(last_buf, c, (GRID_B - 1) * MB).wait()


_mm_call = pl.pallas_call(
    _mm_body,
    grid=(GRID_B,),
    in_specs=[
        pl.BlockSpec((N_ROWS, EMBED), lambda j: (0, 0)),
        pl.BlockSpec((WPAD, EMBED), lambda j: (0, 0)),
        pl.BlockSpec((1, VOCAB), lambda j: (0, 0)),
    ],
    out_specs=pl.BlockSpec(memory_space=pltpu.MemorySpace.HBM),
    out_shape=jax.ShapeDtypeStruct((B, VOCAB), jnp.float32),
    scratch_shapes=[
        pltpu.VMEM((2, MB, VOCAB), jnp.float32),
        pltpu.SemaphoreType.DMA((2, NCH)),
    ],
    compiler_params=pltpu.CompilerParams(
        dimension_semantics=("arbitrary",),
        vmem_limit_bytes=61 * 1024 * 1024,
    ),
)


def kernel(inputs_, table, W, b):
    idx = inputs_.astype(jnp.int32).reshape(NW, N_CHUNKS, CHUNK)
    emb = _make_sc_gather()(idx, table)
    w_bf = jnp.pad(W.astype(jnp.bfloat16), ((0, WPAD - VOCAB), (0, 0)))
    return _mm_call(emb, w_bf, b.reshape(1, VOCAB))


# speedup vs baseline: 1.2256x; 1.2256x over previous
import functools

import jax
import jax.numpy as jnp
from jax import lax
from jax.experimental import pallas as pl
from jax.experimental.pallas import tpu as pltpu
from jax.experimental.pallas import tpu_sc as plsc

VOCAB = 100000
EMBED = 64
MAX_NORM = 1.0
B = 1024
L = 20
N_ROWS = B * L

_NC, _NS = 2, 16
NW = _NC * _NS
ROWS_PER_W = N_ROWS // NW
CHUNK = 128
N_CHUNKS = ROWS_PER_W // CHUNK

VT = 2048
GRID = -(-VOCAB // VT)
TAIL = VOCAB - (GRID - 1) * VT
TAIL_ALN = (TAIL // 128) * 128
TAIL_REM = TAIL - TAIL_ALN
NCH = 4
RCH = B // NCH


@functools.lru_cache(maxsize=1)
def _make_sc_gather():
    @functools.partial(
        pl.kernel,
        mesh=plsc.VectorSubcoreMesh(core_axis_name="c", subcore_axis_name="s"),
        out_type=jax.ShapeDtypeStruct((N_ROWS, EMBED), jnp.float32),
        scratch_types=[
            pltpu.VMEM((N_CHUNKS, CHUNK), jnp.int32),
            pltpu.VMEM((ROWS_PER_W, EMBED), jnp.float32),
            pltpu.SemaphoreType.DMA,
        ],
        compiler_params=pltpu.CompilerParams(use_tc_tiling_on_sc=False),
    )
    def _sc_gather(idx_hbm, table_hbm, emb_hbm, idx_v, rows_v, sem):
        wid = lax.axis_index("s") * _NC + lax.axis_index("c")
        pltpu.sync_copy(idx_hbm.at[wid], idx_v)
        copies = []
        for k in range(N_CHUNKS):
            copies.append(
                pltpu.async_copy(
                    table_hbm.at[idx_v.at[k]],
                    rows_v.at[pl.ds(k * CHUNK, CHUNK)],
                    sem,
                )
            )
        for c in copies:
            c.wait()
        pltpu.sync_copy(rows_v, emb_hbm.at[pl.ds(wid * ROWS_PER_W, ROWS_PER_W)])

    return _sc_gather


def _renorm_body(emb_ref, x_ref):
    e = emb_ref[...]
    ss = jnp.sum(e * e, axis=1, keepdims=True)
    scale = jnp.minimum(1.0, MAX_NORM / jnp.maximum(jnp.sqrt(ss), 1e-7))
    x_ref[...] = jnp.sum((e * scale).reshape(B, L, EMBED), axis=1) * (1.0 / L)


_renorm_call = pl.pallas_call(
    _renorm_body,
    out_shape=jax.ShapeDtypeStruct((B, EMBED), jnp.float32),
)


def _mm_body(x_ref, w_ref, b_ref, out_hbm, out_vmem, rem_vmem, sems):
    i = pl.program_id(0)
    xb = x_ref[...].astype(jnp.bfloat16)
    wb = w_ref[...].astype(jnp.bfloat16)
    acc = lax.dot_general(
        xb, wb, (((1,), (1,)), ((), ())),
        preferred_element_type=jnp.float32,
    ) + b_ref[...]

    def chunk_copy(buf, c, col0, width):
        return pltpu.make_async_copy(
            out_vmem.at[buf, pl.ds(c * RCH, RCH), pl.ds(0, width)],
            out_hbm.at[pl.ds(c * RCH, RCH), pl.ds(col0, width)],
            sems.at[buf, c],
        )

    def rem_copy(buf):
        return pltpu.make_async_copy(
            rem_vmem.at[buf],
            out_hbm.at[:, pl.ds((GRID - 1) * VT + TAIL_ALN, TAIL_REM)],
            sems.at[buf, NCH],
        )

    for buf in range(2):
        @pl.when(lax.rem(i, 2) == buf)
        def _(buf=buf):
            @pl.when(i >= 2)
            def _():
                for c in range(NCH):
                    chunk_copy(buf, c, (i - 2) * VT, VT).wait()

            out_vmem[buf] = acc

            @pl.when(i == GRID - 1)
            def _():
                rem_vmem[buf] = acc[:, TAIL_ALN:TAIL_ALN + TAIL_REM]

            @pl.when(i < GRID - 1)
            def _():
                for c in range(NCH):
                    chunk_copy(buf, c, i * VT, VT).start()

            @pl.when(i == GRID - 1)
            def _():
                for c in range(NCH):
                    chunk_copy(buf, c, i * VT, TAIL_ALN).start()
                rem_copy(buf).start()

    @pl.when(i == GRID - 1)
    def _():
        last_buf = (GRID - 1) % 2
        for c in range(NCH):
            chunk_copy(1 - last_buf, c, (GRID - 2) * VT, VT).wait()
        for c in range(NCH):
            chunk_copy(last_buf, c, (GRID - 1) * VT, TAIL_ALN).wait()
        rem_copy(last_buf).wait()


_mm_call = pl.pallas_call(
    _mm_body,
    grid=(GRID,),
    in_specs=[
        pl.BlockSpec((B, EMBED), lambda i: (0, 0)),
        pl.BlockSpec((VT, EMBED), lambda i: (i, 0)),
        pl.BlockSpec((1, VT), lambda i: (0, i)),
    ],
    out_specs=pl.BlockSpec(memory_space=pltpu.MemorySpace.HBM),
    out_shape=jax.ShapeDtypeStruct((B, VOCAB), jnp.float32),
    scratch_shapes=[
        pltpu.VMEM((2, B, VT), jnp.float32),
        pltpu.VMEM((2, B, TAIL_REM), jnp.float32),
        pltpu.SemaphoreType.DMA((2, NCH + 1)),
    ],
    compiler_params=pltpu.CompilerParams(
        dimension_semantics=("arbitrary",),
    ),
)


def kernel(inputs_, table, W, b):
    idx = inputs_.astype(jnp.int32).reshape(NW, N_CHUNKS, CHUNK)
    emb = _make_sc_gather()(idx, table)
    x = _renorm_call(emb)
    return _mm_call(x, W, b.reshape(1, VOCAB))
